# baseline (speedup 1.0000x reference)
import jax
import jax.numpy as jnp
from jax import lax
from jax.experimental import pallas as pl
from jax.experimental.pallas import tpu as pltpu
from jax.experimental.pallas import tpu_sc as plsc

B, L, H = 1024, 200, 3
D = 32
N = B * L
NC, NS, LANES = 2, 16, 16
NW = NC * NS
ROWS_W = N // NW
CHUNK = 128
NIT = ROWS_W // CHUNK
V = 1000000

TCB = 2048
TGRID = (V + TCB - 1) // TCB
YR = TGRID * (TCB // 4)


def _tc_body(x0, x1, x2, y0, y1, y2):
    eye = jnp.eye(D, dtype=jnp.float32)
    for x, y in ((x0, y0), (x1, y1), (x2, y2)):
        xv = x[...]
        bt = lax.dot_general(xv, eye, (((0,), (0,)), ((), ())),
                             precision=lax.Precision.HIGHEST,
                             preferred_element_type=jnp.float32)
        for a in range(TCB // 128):
            q, p = a // 4, a % 4
            y[128 * q:128 * (q + 1), 32 * p:32 * (p + 1)] = (
                bt[128 * a:128 * (a + 1), :])


def _tc_repack(t0t, t1t, t2t):
    in_spec = pl.BlockSpec((32, TCB), lambda j: (0, j))
    out_spec = pl.BlockSpec((TCB // 4, 128), lambda j: (j, 0))
    return pl.pallas_call(
        _tc_body,
        grid=(TGRID,),
        in_specs=[in_spec] * H,
        out_specs=[out_spec] * H,
        out_shape=[jax.ShapeDtypeStruct((YR, 128), jnp.float32)] * H,
    )(t0t, t1t, t2t)


def _gbody(idx0_hbm, idx1_hbm, idx2_hbm, feats_hbm, y0, y1, y2, wb_hbm,
           out_hbm, iv0, iv1, iv2, rv0, rv1, rv2, ov0, ov1, ov2,
           feats_v, wb_v, g0_v, g1_v, g2_v, asm_v, sem):
    wid = lax.axis_index("s") * NC + lax.axis_index("c")

    pltpu.sync_copy(wb_hbm, wb_v)
    w_lo = wb_v[pl.ds(0, LANES)]
    w_hi = wb_v[pl.ds(LANES, LANES)]
    b_lo = wb_v[pl.ds(2 * LANES, LANES)]
    b_hi = wb_v[pl.ds(3 * LANES, LANES)]

    def iteration(it, carry):
        base = wid * ROWS_W + it * CHUNK

        pltpu.sync_copy(feats_hbm.at[pl.ds(base, CHUNK)], feats_v)
        pltpu.sync_copy(idx0_hbm.at[pl.ds(base, CHUNK)], iv0)
        pltpu.sync_copy(idx1_hbm.at[pl.ds(base, CHUNK)], iv1)
        pltpu.sync_copy(idx2_hbm.at[pl.ds(base, CHUNK)], iv2)

        def rowcalc(j, c):
            for iv, rv, ov in ((iv0, rv0, ov0), (iv1, rv1, ov1),
                               (iv2, rv2, ov2)):
                v = iv[pl.ds(j * LANES, LANES)]
                rv[pl.ds(j * LANES, LANES)] = (
                    (v >> 9) << 7) | (v & 127)
                ov[pl.ds(j * LANES, LANES)] = ((v >> 7) & 3) << 5
            return c
        lax.fori_loop(0, CHUNK // LANES, rowcalc, 0)

        copies = []
        for tbl, rv, gv in ((y0, rv0, g0_v), (y1, rv1, g1_v),
                            (y2, rv2, g2_v)):
            copies.append(pltpu.make_async_copy(
                tbl.at[rv.at[pl.ds(0, CHUNK)]], gv, sem))
        for c in copies:
            c.start()

        def enc(i16, c):
            fvec = feats_v[pl.ds(i16 * LANES, LANES)]
            for k in range(LANES):
                fv = jnp.full((LANES,), fvec[k])
                asm_v[i16 * LANES + k, pl.ds(96, LANES)] = fv * w_lo + b_lo
                asm_v[i16 * LANES + k, pl.ds(112, LANES)] = fv * w_hi + b_hi
            return c
        lax.fori_loop(0, CHUNK // LANES, enc, 0)

        for c in copies:
            c.wait()

        def extract(i16, c):
            o0 = ov0[pl.ds(i16 * LANES, LANES)]
            o1 = ov1[pl.ds(i16 * LANES, LANES)]
            o2 = ov2[pl.ds(i16 * LANES, LANES)]
            for k in range(LANES):
                r = i16 * LANES + k
                for gv, ov, lane0 in ((g0_v, o0, 0), (g1_v, o1, 32),
                                      (g2_v, o2, 64)):
                    off = ov[k]
                    asm_v[r, pl.ds(lane0, LANES)] = gv[r, pl.ds(off, LANES)]
                    asm_v[r, pl.ds(lane0 + LANES, LANES)] = (
                        gv[r, pl.ds(off + LANES, LANES)])
            return c
        lax.fori_loop(0, CHUNK // LANES, extract, 0)

        pltpu.sync_copy(asm_v, out_hbm.at[pl.ds(base, CHUNK), :])
        return carry

    lax.fori_loop(0, NIT, iteration, 0)


def _sc_embed(idx0, idx1, idx2, feats, y0, y1, y2, wb):
    mesh = plsc.VectorSubcoreMesh(core_axis_name="c", subcore_axis_name="s",
                                  num_cores=NC, num_subcores=NS)
    f = pl.kernel(
        _gbody,
        out_type=jax.ShapeDtypeStruct((N, 4 * D), jnp.float32),
        mesh=mesh,
        compiler_params=pltpu.CompilerParams(use_tc_tiling_on_sc=True),
        scratch_types=[
            pltpu.VMEM((CHUNK,), jnp.int32),
            pltpu.VMEM((CHUNK,), jnp.int32),
            pltpu.VMEM((CHUNK,), jnp.int32),
            pltpu.VMEM((CHUNK,), jnp.int32),
            pltpu.VMEM((CHUNK,), jnp.int32),
            pltpu.VMEM((CHUNK,), jnp.int32),
            pltpu.VMEM((CHUNK,), jnp.int32),
            pltpu.VMEM((CHUNK,), jnp.int32),
            pltpu.VMEM((CHUNK,), jnp.int32),
            pltpu.VMEM((CHUNK,), jnp.float32),
            pltpu.VMEM((4 * LANES,), jnp.float32),
            pltpu.VMEM((CHUNK, 128), jnp.float32),
            pltpu.VMEM((CHUNK, 128), jnp.float32),
            pltpu.VMEM((CHUNK, 128), jnp.float32),
            pltpu.VMEM((CHUNK, 128), jnp.float32),
            pltpu.SemaphoreType.DMA,
        ],
    )
    return f(idx0, idx1, idx2, feats, y0, y1, y2, wb)


@jax.jit
def _run(tokens, features, T0, T1, T2, W, b):
    tok = tokens.reshape(N, H)
    feats = features.reshape(N)
    wb = jnp.concatenate([W.reshape(D), b.reshape(D)])
    y0, y1, y2 = _tc_repack(T0.T, T1.T, T2.T)
    out = _sc_embed(tok[:, 0], tok[:, 1], tok[:, 2], feats, y0, y1, y2, wb)
    return out.reshape(B, L, (H + 1) * D)


def kernel(tokens, features, T0, T1, T2, W, b):
    return _run(tokens, features, T0, T1, T2, W, b)


# speedup vs baseline: 1.5927x; 1.5927x over previous
import jax
import jax.numpy as jnp
from jax import lax
from jax.experimental import pallas as pl
from jax.experimental.pallas import tpu as pltpu
from jax.experimental.pallas import tpu_sc as plsc

B, L, H = 1024, 200, 3
D = 32
N = B * L
NC, NS, LANES = 2, 16, 16
NW = NC * NS
ROWS_W = N // NW
CHUNK = 128
NIT = ROWS_W // CHUNK
V = 1000000

TCB = 2048
TGRID = (V + TCB - 1) // TCB
YR = TGRID * (TCB // 4)


def _tc_body(x0, x1, x2, y0, y1, y2):
    eye = jnp.eye(D, dtype=jnp.float32)
    for x, y in ((x0, y0), (x1, y1), (x2, y2)):
        xv = x[...]
        bt = lax.dot_general(xv, eye, (((0,), (0,)), ((), ())),
                             preferred_element_type=jnp.float32)
        for a in range(TCB // 128):
            q, p = a // 4, a % 4
            y[128 * q:128 * (q + 1), 32 * p:32 * (p + 1)] = (
                bt[128 * a:128 * (a + 1), :])


def _tc_repack(t0t, t1t, t2t):
    in_spec = pl.BlockSpec((32, TCB), lambda j: (0, j))
    out_spec = pl.BlockSpec((TCB // 4, 128), lambda j: (j, 0))
    return pl.pallas_call(
        _tc_body,
        grid=(TGRID,),
        in_specs=[in_spec] * H,
        out_specs=[out_spec] * H,
        out_shape=[jax.ShapeDtypeStruct((YR, 128), jnp.float32)] * H,
    )(t0t, t1t, t2t)


def _gbody(idx0_hbm, idx1_hbm, idx2_hbm, feats_hbm, y0, y1, y2, wb_hbm,
           out_hbm, iv0, iv1, iv2, rv0, rv1, rv2, ov0, ov1, ov2,
           feats_v, wb_v, g0_v, g1_v, g2_v, asm_v, sem):
    wid = lax.axis_index("s") * NC + lax.axis_index("c")

    pltpu.sync_copy(wb_hbm, wb_v)
    w_lo = wb_v[pl.ds(0, LANES)]
    w_hi = wb_v[pl.ds(LANES, LANES)]
    b_lo = wb_v[pl.ds(2 * LANES, LANES)]
    b_hi = wb_v[pl.ds(3 * LANES, LANES)]

    def iteration(it, carry):
        base = wid * ROWS_W + it * CHUNK

        pltpu.sync_copy(feats_hbm.at[pl.ds(base, CHUNK)], feats_v)
        pltpu.sync_copy(idx0_hbm.at[pl.ds(base, CHUNK)], iv0)
        pltpu.sync_copy(idx1_hbm.at[pl.ds(base, CHUNK)], iv1)
        pltpu.sync_copy(idx2_hbm.at[pl.ds(base, CHUNK)], iv2)

        def rowcalc(j, c):
            for iv, rv, ov in ((iv0, rv0, ov0), (iv1, rv1, ov1),
                               (iv2, rv2, ov2)):
                v = iv[pl.ds(j * LANES, LANES)]
                rv[pl.ds(j * LANES, LANES)] = (
                    (v >> 9) << 7) | (v & 127)
                ov[pl.ds(j * LANES, LANES)] = ((v >> 7) & 3) << 5
            return c
        lax.fori_loop(0, CHUNK // LANES, rowcalc, 0)

        copies = []
        for tbl, rv, gv in ((y0, rv0, g0_v), (y1, rv1, g1_v),
                            (y2, rv2, g2_v)):
            copies.append(pltpu.make_async_copy(
                tbl.at[rv.at[pl.ds(0, CHUNK)]], gv, sem))
        for c in copies:
            c.start()

        def enc(i16, c):
            fvec = feats_v[pl.ds(i16 * LANES, LANES)]
            for k in range(LANES):
                fv = jnp.full((LANES,), fvec[k])
                asm_v[i16 * LANES + k, pl.ds(96, LANES)] = fv * w_lo + b_lo
                asm_v[i16 * LANES + k, pl.ds(112, LANES)] = fv * w_hi + b_hi
            return c
        lax.fori_loop(0, CHUNK // LANES, enc, 0)

        for c in copies:
            c.wait()

        def extract(i16, c):
            o0 = ov0[pl.ds(i16 * LANES, LANES)]
            o1 = ov1[pl.ds(i16 * LANES, LANES)]
            o2 = ov2[pl.ds(i16 * LANES, LANES)]
            for k in range(LANES):
                r = i16 * LANES + k
                for gv, ov, lane0 in ((g0_v, o0, 0), (g1_v, o1, 32),
                                      (g2_v, o2, 64)):
                    off = ov[k]
                    asm_v[r, pl.ds(lane0, LANES)] = gv[r, pl.ds(off, LANES)]
                    asm_v[r, pl.ds(lane0 + LANES, LANES)] = (
                        gv[r, pl.ds(off + LANES, LANES)])
            return c
        lax.fori_loop(0, CHUNK // LANES, extract, 0)

        pltpu.sync_copy(asm_v, out_hbm.at[pl.ds(base, CHUNK), :])
        return carry

    lax.fori_loop(0, NIT, iteration, 0)


def _sc_embed(idx0, idx1, idx2, feats, y0, y1, y2, wb):
    mesh = plsc.VectorSubcoreMesh(core_axis_name="c", subcore_axis_name="s",
                                  num_cores=NC, num_subcores=NS)
    f = pl.kernel(
        _gbody,
        out_type=jax.ShapeDtypeStruct((N, 4 * D), jnp.float32),
        mesh=mesh,
        compiler_params=pltpu.CompilerParams(use_tc_tiling_on_sc=True),
        scratch_types=[
            pltpu.VMEM((CHUNK,), jnp.int32),
            pltpu.VMEM((CHUNK,), jnp.int32),
            pltpu.VMEM((CHUNK,), jnp.int32),
            pltpu.VMEM((CHUNK,), jnp.int32),
            pltpu.VMEM((CHUNK,), jnp.int32),
            pltpu.VMEM((CHUNK,), jnp.int32),
            pltpu.VMEM((CHUNK,), jnp.int32),
            pltpu.VMEM((CHUNK,), jnp.int32),
            pltpu.VMEM((CHUNK,), jnp.int32),
            pltpu.VMEM((CHUNK,), jnp.float32),
            pltpu.VMEM((4 * LANES,), jnp.float32),
            pltpu.VMEM((CHUNK, 128), jnp.float32),
            pltpu.VMEM((CHUNK, 128), jnp.float32),
            pltpu.VMEM((CHUNK, 128), jnp.float32),
            pltpu.VMEM((CHUNK, 128), jnp.float32),
            pltpu.SemaphoreType.DMA,
        ],
    )
    return f(idx0, idx1, idx2, feats, y0, y1, y2, wb)


@jax.jit
def _run(tokens, features, T0, T1, T2, W, b):
    tok = tokens.reshape(N, H)
    feats = features.reshape(N)
    wb = jnp.concatenate([W.reshape(D), b.reshape(D)])
    y0, y1, y2 = _tc_repack(T0.T, T1.T, T2.T)
    out = _sc_embed(tok[:, 0], tok[:, 1], tok[:, 2], feats, y0, y1, y2, wb)
    return out.reshape(B, L, (H + 1) * D)


def kernel(tokens, features, T0, T1, T2, W, b):
    return _run(tokens, features, T0, T1, T2, W, b)


# speedup vs baseline: 1.7164x; 1.0777x over previous
import jax
import jax.numpy as jnp
from jax import lax
from jax.experimental import pallas as pl
from jax.experimental.pallas import tpu as pltpu
from jax.experimental.pallas import tpu_sc as plsc

B, L, H = 1024, 200, 3
D = 32
N = B * L
NC, NS, LANES = 2, 16, 16
NW = NC * NS
ROWS_W = N // NW
CHUNK = 128
NIT = ROWS_W // CHUNK
V = 1000000

TCB = 2048
TGRID = (V + TCB - 1) // TCB
YR = TGRID * TCB


def _tc_body(x0, x1, x2, y0, y1, y2):
    eye = jnp.tile(jnp.eye(D, dtype=jnp.float32), (1, 4))
    for x, y in ((x0, y0), (x1, y1), (x2, y2)):
        xv = x[...]
        y[...] = lax.dot_general(xv, eye, (((0,), (0,)), ((), ())),
                                 preferred_element_type=jnp.float32)


def _tc_repack(t0t, t1t, t2t):
    in_spec = pl.BlockSpec((32, TCB), lambda j: (0, j))
    out_spec = pl.BlockSpec((TCB, 128), lambda j: (j, 0))
    return pl.pallas_call(
        _tc_body,
        grid=(TGRID,),
        in_specs=[in_spec] * H,
        out_specs=[out_spec] * H,
        out_shape=[jax.ShapeDtypeStruct((YR, 128), jnp.float32)] * H,
    )(t0t, t1t, t2t)


def _gbody(idx0_hbm, idx1_hbm, idx2_hbm, feats_hbm, y0, y1, y2, wb_hbm,
           out_hbm, iv0, iv1, iv2,
           feats_v, wb_v, g0_v, g1_v, g2_v, asm_v, sem):
    wid = lax.axis_index("s") * NC + lax.axis_index("c")

    pltpu.sync_copy(wb_hbm, wb_v)
    w_lo = wb_v[pl.ds(0, LANES)]
    w_hi = wb_v[pl.ds(LANES, LANES)]
    b_lo = wb_v[pl.ds(2 * LANES, LANES)]
    b_hi = wb_v[pl.ds(3 * LANES, LANES)]

    def iteration(it, carry):
        base = wid * ROWS_W + it * CHUNK

        pltpu.sync_copy(feats_hbm.at[pl.ds(base, CHUNK)], feats_v)
        pltpu.sync_copy(idx0_hbm.at[pl.ds(base, CHUNK)], iv0)
        pltpu.sync_copy(idx1_hbm.at[pl.ds(base, CHUNK)], iv1)
        pltpu.sync_copy(idx2_hbm.at[pl.ds(base, CHUNK)], iv2)

        copies = []
        for tbl, iv, gv in ((y0, iv0, g0_v), (y1, iv1, g1_v),
                            (y2, iv2, g2_v)):
            copies.append(pltpu.make_async_copy(
                tbl.at[iv.at[pl.ds(0, CHUNK)]], gv, sem))
        for c in copies:
            c.start()

        def enc(i16, c):
            fvec = feats_v[pl.ds(i16 * LANES, LANES)]
            for k in range(LANES):
                fv = jnp.full((LANES,), fvec[k])
                asm_v[i16 * LANES + k, pl.ds(96, LANES)] = fv * w_lo + b_lo
                asm_v[i16 * LANES + k, pl.ds(112, LANES)] = fv * w_hi + b_hi
            return c
        lax.fori_loop(0, CHUNK // LANES, enc, 0)

        for c in copies:
            c.wait()

        def extract(i16, c):
            for k in range(LANES):
                r = i16 * LANES + k
                for gv, lane0 in ((g0_v, 0), (g1_v, 32), (g2_v, 64)):
                    asm_v[r, pl.ds(lane0, LANES)] = gv[r, pl.ds(0, LANES)]
                    asm_v[r, pl.ds(lane0 + LANES, LANES)] = (
                        gv[r, pl.ds(LANES, LANES)])
            return c
        lax.fori_loop(0, CHUNK // LANES, extract, 0)

        pltpu.sync_copy(asm_v, out_hbm.at[pl.ds(base, CHUNK), :])
        return carry

    lax.fori_loop(0, NIT, iteration, 0)


def _sc_embed(idx0, idx1, idx2, feats, y0, y1, y2, wb):
    mesh = plsc.VectorSubcoreMesh(core_axis_name="c", subcore_axis_name="s",
                                  num_cores=NC, num_subcores=NS)
    f = pl.kernel(
        _gbody,
        out_type=jax.ShapeDtypeStruct((N, 4 * D), jnp.float32),
        mesh=mesh,
        compiler_params=pltpu.CompilerParams(use_tc_tiling_on_sc=True),
        scratch_types=[
            pltpu.VMEM((CHUNK,), jnp.int32),
            pltpu.VMEM((CHUNK,), jnp.int32),
            pltpu.VMEM((CHUNK,), jnp.int32),
            pltpu.VMEM((CHUNK,), jnp.float32),
            pltpu.VMEM((4 * LANES,), jnp.float32),
            pltpu.VMEM((CHUNK, 128), jnp.float32),
            pltpu.VMEM((CHUNK, 128), jnp.float32),
            pltpu.VMEM((CHUNK, 128), jnp.float32),
            pltpu.VMEM((CHUNK, 128), jnp.float32),
            pltpu.SemaphoreType.DMA,
        ],
    )
    return f(idx0, idx1, idx2, feats, y0, y1, y2, wb)


@jax.jit
def _run(tokens, features, T0, T1, T2, W, b):
    tok = tokens.reshape(N, H)
    feats = features.reshape(N)
    wb = jnp.concatenate([W.reshape(D), b.reshape(D)])
    y0, y1, y2 = _tc_repack(T0.T, T1.T, T2.T)
    out = _sc_embed(tok[:, 0], tok[:, 1], tok[:, 2], feats, y0, y1, y2, wb)
    return out.reshape(B, L, (H + 1) * D)


def kernel(tokens, features, T0, T1, T2, W, b):
    return _run(tokens, features, T0, T1, T2, W, b)
